# Initial kernel scaffold; baseline (speedup 1.0000x reference)
import jax
import jax.numpy as jnp
from jax.experimental import pallas as pl


def kernel(x, target):
    raise NotImplementedError("write your pallas kernel here")


# speedup vs baseline: 7.7531x; 7.7531x over previous
import jax
import jax.numpy as jnp
from jax.experimental import pallas as pl
from jax.experimental.pallas import tpu as pltpu

_BINS = 10
_ALPHA = 0.75
_ROWS = 65536
_COLS = 80
_N = float(_ROWS * _COLS)
_BLOCK_ROWS = 4096


def _ghm_kernel(x_ref, t_ref, out_ref, acc_ref):
    i = pl.program_id(0)
    nblk = pl.num_programs(0)

    @pl.when(i == 0)
    def _init():
        acc_ref[...] = jnp.zeros_like(acc_ref)

    x = x_ref[...]
    t = t_ref[...]

    e = jnp.exp(-jnp.abs(x))
    r = 1.0 / (1.0 + e)
    sig = jnp.where(x >= 0.0, r, e * r)
    g = jnp.abs(sig - t)
    binf = jnp.floor(g * (_BINS - 0.0001))
    bce = jnp.maximum(x, 0.0) - x * t + jnp.log1p(e)

    for b in range(_BINS):
        m = (binf == float(b)).astype(jnp.float32)
        acc_ref[b, :] += jnp.sum(m, axis=0)
        acc_ref[_BINS + b, :] += jnp.sum(m * bce, axis=0)

    @pl.when(i == nblk - 1)
    def _epilogue():
        counts = jnp.sum(acc_ref[0:_BINS, :], axis=1, keepdims=True)
        sums = jnp.sum(acc_ref[_BINS:2 * _BINS, :], axis=1, keepdims=True)
        acc_sum = (1.0 - _ALPHA) * counts
        w = jnp.where(counts >= 1.0, _N / jnp.maximum(acc_sum, 1e-12), 0.0)
        nonempty = jnp.sum((counts >= 1.0).astype(jnp.float32))
        w = w / jnp.maximum(nonempty, 1.0)
        w = jnp.maximum(w, 0.0001)
        total = jnp.sum(w * sums, axis=0, keepdims=True)
        out_ref[...] = total / _N


def kernel(x, target):
    grid = (_ROWS // _BLOCK_ROWS,)
    out = pl.pallas_call(
        _ghm_kernel,
        grid=grid,
        in_specs=[
            pl.BlockSpec((_BLOCK_ROWS, _COLS), lambda i: (i, 0)),
            pl.BlockSpec((_BLOCK_ROWS, _COLS), lambda i: (i, 0)),
        ],
        out_specs=pl.BlockSpec((1, 1), lambda i: (0, 0)),
        out_shape=jax.ShapeDtypeStruct((1, 1), jnp.float32),
        scratch_shapes=[pltpu.VMEM((2 * _BINS, _COLS), jnp.float32)],
    )(x, target)
    return out[0, 0]


# speedup vs baseline: 8.4448x; 1.0892x over previous
import jax
import jax.numpy as jnp
from jax.experimental import pallas as pl
from jax.experimental.pallas import tpu as pltpu

_BINS = 10
_ALPHA = 0.75
_ROWS = 65536
_COLS = 80
_N = float(_ROWS * _COLS)
_BLOCK_ROWS = 4096
_CH = 32
_NEG_L2E = -1.4426950408889634
_NEG_LN2 = -0.6931471805599453
_THRESH = [float(b) / 9.9999 for b in range(1, _BINS)]


def _fold(v):
    return jnp.sum(v.reshape(_CH // 8, 8, _COLS), axis=0)


def _ghm_kernel(x_ref, t_ref, out_ref, acc_ref):
    i = pl.program_id(0)
    nblk = pl.num_programs(0)

    zero = jnp.zeros((8, _COLS), jnp.float32)

    def body(j, carry):
        utot, cnts, sums = carry
        base = j * _CH
        x = x_ref[pl.ds(base, _CH), :]
        t = t_ref[pl.ds(base, _CH), :]
        e = jnp.exp2(jnp.abs(x) * _NEG_L2E)
        r = 1.0 / (1.0 + e)
        er = e * r
        swap = (x >= 0.0) == (t >= 0.5)
        g = jnp.where(swap, er, r)
        u = jnp.log2(1.0 - g)
        utot = utot + _fold(u)
        new_c = []
        new_s = []
        for k in range(_BINS - 1):
            m = jnp.where(g >= _THRESH[k], 1.0, 0.0)
            new_c.append(cnts[k] + _fold(m))
            new_s.append(sums[k] + _fold(m * u))
        return utot, tuple(new_c), tuple(new_s)

    init = (zero, (zero,) * (_BINS - 1), (zero,) * (_BINS - 1))
    utot, cnts, sums = jax.lax.fori_loop(0, _BLOCK_ROWS // _CH, body, init)

    flat = jnp.stack(list(cnts) + list(sums) + [utot], axis=0)

    @pl.when(i == 0)
    def _init():
        acc_ref[...] = flat

    @pl.when(i > 0)
    def _accum():
        acc_ref[...] += flat

    @pl.when(i == nblk - 1)
    def _epilogue():
        a = jnp.sum(jnp.sum(acc_ref[...], axis=2), axis=1, keepdims=True)
        cge = a[0:_BINS - 1]
        sge = a[_BINS - 1:2 * _BINS - 2]
        ut = a[2 * _BINS - 2:2 * _BINS - 1]
        counts = jnp.concatenate(
            [_N - cge[0:1], cge[:-1] - cge[1:], cge[-1:]], axis=0)
        usums = jnp.concatenate(
            [ut - sge[0:1], sge[:-1] - sge[1:], sge[-1:]], axis=0)
        bsums = _NEG_LN2 * usums
        acc_sum = (1.0 - _ALPHA) * counts
        w = jnp.where(counts >= 1.0, _N / jnp.maximum(acc_sum, 1e-12), 0.0)
        nonempty = jnp.sum((counts >= 1.0).astype(jnp.float32))
        w = w / jnp.maximum(nonempty, 1.0)
        w = jnp.maximum(w, 0.0001)
        total = jnp.sum(w * bsums, axis=0, keepdims=True)
        out_ref[...] = total / _N


def kernel(x, target):
    grid = (_ROWS // _BLOCK_ROWS,)
    out = pl.pallas_call(
        _ghm_kernel,
        grid=grid,
        in_specs=[
            pl.BlockSpec((_BLOCK_ROWS, _COLS), lambda i: (i, 0)),
            pl.BlockSpec((_BLOCK_ROWS, _COLS), lambda i: (i, 0)),
        ],
        out_specs=pl.BlockSpec((1, 1), lambda i: (0, 0)),
        out_shape=jax.ShapeDtypeStruct((1, 1), jnp.float32),
        scratch_shapes=[pltpu.VMEM((2 * _BINS - 1, 8, _COLS), jnp.float32)],
    )(x, target)
    return out[0, 0]


# speedup vs baseline: 8.4964x; 1.0061x over previous
import jax
import jax.numpy as jnp
from jax.experimental import pallas as pl
from jax.experimental.pallas import tpu as pltpu

_BINS = 10
_ALPHA = 0.75
_ROWS = 65536
_COLS = 80
_N = float(_ROWS * _COLS)
_BLOCK_ROWS = 4096
_CH = 32
_NEG_L2E = -1.4426950408889634
_NEG_LN2 = -0.6931471805599453
_QTHRESH = [1.0 - float(b) / 9.9999 for b in range(1, _BINS)]


def _fold(v):
    return jnp.sum(v.reshape(_CH // 8, 8, _COLS), axis=0)


def _ghm_kernel(x_ref, t_ref, out_ref, acc_ref):
    i = pl.program_id(0)
    nblk = pl.num_programs(0)

    zero = jnp.zeros((8, _COLS), jnp.float32)

    def body(j, carry):
        utot, cnts, sums = carry
        base = j * _CH
        x = x_ref[pl.ds(base, _CH), :]
        t = t_ref[pl.ds(base, _CH), :]
        e = jnp.exp2(jnp.abs(x) * _NEG_L2E)
        r = 1.0 / (1.0 + e)
        er = e * r
        xs = x * (1.0 - 2.0 * t)
        q = jnp.where(xs >= 0.0, er, r)
        u = jnp.log2(q)
        utot = utot + _fold(u)
        new_c = []
        new_s = []
        for k in range(_BINS - 1):
            mask = q <= _QTHRESH[k]
            new_c.append(cnts[k] + _fold(jnp.where(mask, 1.0, 0.0)))
            new_s.append(sums[k] + _fold(jnp.where(mask, u, 0.0)))
        return utot, tuple(new_c), tuple(new_s)

    init = (zero, (zero,) * (_BINS - 1), (zero,) * (_BINS - 1))
    utot, cnts, sums = jax.lax.fori_loop(0, _BLOCK_ROWS // _CH, body, init)

    flat = jnp.stack(list(cnts) + list(sums) + [utot], axis=0)

    @pl.when(i == 0)
    def _init():
        acc_ref[...] = flat

    @pl.when(i > 0)
    def _accum():
        acc_ref[...] += flat

    @pl.when(i == nblk - 1)
    def _epilogue():
        a = jnp.sum(jnp.sum(acc_ref[...], axis=2), axis=1, keepdims=True)
        cge = a[0:_BINS - 1]
        sge = a[_BINS - 1:2 * _BINS - 2]
        ut = a[2 * _BINS - 2:2 * _BINS - 1]
        counts = jnp.concatenate(
            [_N - cge[0:1], cge[:-1] - cge[1:], cge[-1:]], axis=0)
        usums = jnp.concatenate(
            [ut - sge[0:1], sge[:-1] - sge[1:], sge[-1:]], axis=0)
        bsums = _NEG_LN2 * usums
        acc_sum = (1.0 - _ALPHA) * counts
        w = jnp.where(counts >= 1.0, _N / jnp.maximum(acc_sum, 1e-12), 0.0)
        nonempty = jnp.sum((counts >= 1.0).astype(jnp.float32))
        w = w / jnp.maximum(nonempty, 1.0)
        w = jnp.maximum(w, 0.0001)
        total = jnp.sum(w * bsums, axis=0, keepdims=True)
        out_ref[...] = total / _N


def kernel(x, target):
    grid = (_ROWS // _BLOCK_ROWS,)
    out = pl.pallas_call(
        _ghm_kernel,
        grid=grid,
        in_specs=[
            pl.BlockSpec((_BLOCK_ROWS, _COLS), lambda i: (i, 0)),
            pl.BlockSpec((_BLOCK_ROWS, _COLS), lambda i: (i, 0)),
        ],
        out_specs=pl.BlockSpec((1, 1), lambda i: (0, 0)),
        out_shape=jax.ShapeDtypeStruct((1, 1), jnp.float32),
        scratch_shapes=[pltpu.VMEM((2 * _BINS - 1, 8, _COLS), jnp.float32)],
    )(x, target)
    return out[0, 0]


# speedup vs baseline: 11.0300x; 1.2982x over previous
import jax
import jax.numpy as jnp
from jax.experimental import pallas as pl
from jax.experimental.pallas import tpu as pltpu

_BINS = 10
_ALPHA = 0.75
_ROWS = 65536
_COLS = 80
_N = float(_ROWS * _COLS)
_BLOCK_ROWS = 4096
_CH = 64
_NEG_L2E = -1.4426950408889634
_NEG_LN2 = -0.6931471805599453
_QTHRESH = [1.0 - float(b) / 9.9999 for b in range(1, _BINS)]


def _fold(v):
    w = v.reshape(_CH // 8, 8, _COLS)
    out = w[0]
    for k in range(1, _CH // 8):
        out = out + w[k]
    return out


def _ghm_kernel(x_ref, t_ref, out_ref, acc_ref):
    i = pl.program_id(0)
    nblk = pl.num_programs(0)

    zero = jnp.zeros((8, _COLS), jnp.float32)

    def body(j, carry):
        utot, cnts, sums = carry
        base = j * _CH
        x = x_ref[pl.ds(base, _CH), :]
        t = t_ref[pl.ds(base, _CH), :]
        e = jnp.exp2(jnp.abs(x) * _NEG_L2E)
        r = 1.0 / (1.0 + e)
        er = e * r
        xs = x * (1.0 - 2.0 * t)
        q = jnp.where(xs >= 0.0, er, r)
        u = jnp.log2(q)
        utot = utot + _fold(u)
        new_c = []
        new_s = []
        for k in range(_BINS - 1):
            mask = q <= _QTHRESH[k]
            new_c.append(cnts[k] + _fold(jnp.where(mask, 1.0, 0.0)))
            new_s.append(sums[k] + _fold(jnp.where(mask, u, 0.0)))
        return utot, tuple(new_c), tuple(new_s)

    init = (zero, (zero,) * (_BINS - 1), (zero,) * (_BINS - 1))
    utot, cnts, sums = jax.lax.fori_loop(0, _BLOCK_ROWS // _CH, body, init)

    flat = jnp.stack(list(cnts) + list(sums) + [utot], axis=0)

    @pl.when(i == 0)
    def _init():
        acc_ref[...] = flat

    @pl.when(i > 0)
    def _accum():
        acc_ref[...] += flat

    @pl.when(i == nblk - 1)
    def _epilogue():
        a = jnp.sum(jnp.sum(acc_ref[...], axis=2), axis=1, keepdims=True)
        cge = a[0:_BINS - 1]
        sge = a[_BINS - 1:2 * _BINS - 2]
        ut = a[2 * _BINS - 2:2 * _BINS - 1]
        counts = jnp.concatenate(
            [_N - cge[0:1], cge[:-1] - cge[1:], cge[-1:]], axis=0)
        usums = jnp.concatenate(
            [ut - sge[0:1], sge[:-1] - sge[1:], sge[-1:]], axis=0)
        bsums = _NEG_LN2 * usums
        acc_sum = (1.0 - _ALPHA) * counts
        w = jnp.where(counts >= 1.0, _N / jnp.maximum(acc_sum, 1e-12), 0.0)
        nonempty = jnp.sum((counts >= 1.0).astype(jnp.float32))
        w = w / jnp.maximum(nonempty, 1.0)
        w = jnp.maximum(w, 0.0001)
        total = jnp.sum(w * bsums, axis=0, keepdims=True)
        out_ref[...] = total / _N


def kernel(x, target):
    grid = (_ROWS // _BLOCK_ROWS,)
    out = pl.pallas_call(
        _ghm_kernel,
        grid=grid,
        in_specs=[
            pl.BlockSpec((_BLOCK_ROWS, _COLS), lambda i: (i, 0)),
            pl.BlockSpec((_BLOCK_ROWS, _COLS), lambda i: (i, 0)),
        ],
        out_specs=pl.BlockSpec((1, 1), lambda i: (0, 0)),
        out_shape=jax.ShapeDtypeStruct((1, 1), jnp.float32),
        scratch_shapes=[pltpu.VMEM((2 * _BINS - 1, 8, _COLS), jnp.float32)],
    )(x, target)
    return out[0, 0]


# speedup vs baseline: 11.9184x; 1.0805x over previous
import jax
import jax.numpy as jnp
from jax.experimental import pallas as pl
from jax.experimental.pallas import tpu as pltpu
import math

_BINS = 10
_ALPHA = 0.75
_ROWS = 65536
_COLS = 80
_N = float(_ROWS * _COLS)
_BLOCK_ROWS = 4096
_CH = 64
_L2E = 1.4426950408889634
_LN2 = 0.6931471805599453
_XTHRESH = [math.log((b / 9.9999) / (1.0 - b / 9.9999)) for b in range(1, _BINS)]


def _fold(v):
    w = v.reshape(_CH // 8, 8, _COLS)
    out = w[0]
    for k in range(1, _CH // 8):
        out = out + w[k]
    return out


def _ghm_kernel(x_ref, t_ref, out_ref, acc_ref):
    i = pl.program_id(0)
    nblk = pl.num_programs(0)

    zero = jnp.zeros((8, _COLS), jnp.float32)

    def body(j, carry):
        utot, cnts, sums = carry
        base = j * _CH
        x = x_ref[pl.ds(base, _CH), :]
        t = t_ref[pl.ds(base, _CH), :]
        xb = jax.lax.bitcast_convert_type(x, jnp.uint32)
        tb = jax.lax.bitcast_convert_type(t, jnp.uint32)
        xs = jax.lax.bitcast_convert_type(xb ^ (tb << jnp.uint32(8)),
                                          jnp.float32)
        p = jnp.exp2(xs * _L2E)
        u = jnp.log2(1.0 + p)
        utot = utot + _fold(u)
        new_c = []
        new_s = []
        for k in range(_BINS - 1):
            mask = xs >= _XTHRESH[k]
            new_c.append(cnts[k] + _fold(jnp.where(mask, 1.0, 0.0)))
            new_s.append(sums[k] + _fold(jnp.where(mask, u, 0.0)))
        return utot, tuple(new_c), tuple(new_s)

    init = (zero, (zero,) * (_BINS - 1), (zero,) * (_BINS - 1))
    utot, cnts, sums = jax.lax.fori_loop(0, _BLOCK_ROWS // _CH, body, init)

    flat = jnp.stack(list(cnts) + list(sums) + [utot], axis=0)

    @pl.when(i == 0)
    def _init():
        acc_ref[...] = flat

    @pl.when(i > 0)
    def _accum():
        acc_ref[...] += flat

    @pl.when(i == nblk - 1)
    def _epilogue():
        a = jnp.sum(jnp.sum(acc_ref[...], axis=2), axis=1, keepdims=True)
        cge = a[0:_BINS - 1]
        sge = a[_BINS - 1:2 * _BINS - 2]
        ut = a[2 * _BINS - 2:2 * _BINS - 1]
        counts = jnp.concatenate(
            [_N - cge[0:1], cge[:-1] - cge[1:], cge[-1:]], axis=0)
        usums = jnp.concatenate(
            [ut - sge[0:1], sge[:-1] - sge[1:], sge[-1:]], axis=0)
        bsums = _LN2 * usums
        acc_sum = (1.0 - _ALPHA) * counts
        w = jnp.where(counts >= 1.0, _N / jnp.maximum(acc_sum, 1e-12), 0.0)
        nonempty = jnp.sum((counts >= 1.0).astype(jnp.float32))
        w = w / jnp.maximum(nonempty, 1.0)
        w = jnp.maximum(w, 0.0001)
        total = jnp.sum(w * bsums, axis=0, keepdims=True)
        out_ref[...] = total / _N


def kernel(x, target):
    grid = (_ROWS // _BLOCK_ROWS,)
    out = pl.pallas_call(
        _ghm_kernel,
        grid=grid,
        in_specs=[
            pl.BlockSpec((_BLOCK_ROWS, _COLS), lambda i: (i, 0)),
            pl.BlockSpec((_BLOCK_ROWS, _COLS), lambda i: (i, 0)),
        ],
        out_specs=pl.BlockSpec((1, 1), lambda i: (0, 0)),
        out_shape=jax.ShapeDtypeStruct((1, 1), jnp.float32),
        scratch_shapes=[pltpu.VMEM((2 * _BINS - 1, 8, _COLS), jnp.float32)],
    )(x, target)
    return out[0, 0]
